# Initial kernel scaffold; baseline (speedup 1.0000x reference)
import functools

import jax
import jax.numpy as jnp
from jax import lax
from jax.experimental import pallas as pl
from jax.experimental.pallas import tpu as pltpu
from jax.experimental.pallas import tpu_sc as plsc

N = 10000
D = 128
E = 320000

NPAD = 10240
K = 128
CHUNKS = 2528
E_PAD = CHUNKS * K
NCORES = 2
NSUB = 16
CH_PER_CORE = CHUNKS // NCORES
CH_PER_TILE = CH_PER_CORE // NSUB
ROWS_PER_TILE = NPAD // NSUB
JUNK_SRC = N + 100
JUNK_DST = N + 200
DEGW = 16

_mesh = plsc.VectorSubcoreMesh(core_axis_name="c", subcore_axis_name="s")


def _fill_rows(ref, nrows, ncolchunks, value):
    v = jnp.full((16,), value, jnp.float32)

    def outer(r, _):
        def inner(k, _):
            ref[r, pl.ds(k * 16, 16)] = v
            return 0
        return lax.fori_loop(0, ncolchunks, inner, 0)

    lax.fori_loop(0, nrows, outer, 0)


def _sc_deg_body(dst_hbm, out_hbm, ones_v, zeros_v, idx_v, deg_sh):
    c = lax.axis_index("c")
    s = lax.axis_index("s")
    _fill_rows(ones_v, K, DEGW // 16, 1.0)
    _fill_rows(zeros_v, K, DEGW // 16, 0.0)
    for z in range(ROWS_PER_TILE // K):
        pltpu.sync_copy(zeros_v, deg_sh.at[pl.ds(s * ROWS_PER_TILE + z * K, K)])
    plsc.subcore_barrier()
    base = c * CH_PER_CORE + s * CH_PER_TILE
    pltpu.sync_copy(dst_hbm.at[pl.ds(base, CH_PER_TILE)], idx_v)

    def body(j, _):
        pltpu.sync_copy(ones_v, deg_sh.at[idx_v.at[j]], add=True)
        return 0

    lax.fori_loop(0, CH_PER_TILE, body, 0)
    plsc.subcore_barrier()
    r0 = s * ROWS_PER_TILE
    pltpu.sync_copy(deg_sh.at[pl.ds(r0, ROWS_PER_TILE)],
                    out_hbm.at[c, pl.ds(r0, ROWS_PER_TILE)])


_sc_deg = functools.partial(
    pl.kernel,
    out_type=jax.ShapeDtypeStruct((NCORES, NPAD, DEGW), jnp.float32),
    mesh=_mesh,
    scratch_types=[
        pltpu.VMEM((K, DEGW), jnp.float32),
        pltpu.VMEM((K, DEGW), jnp.float32),
        pltpu.VMEM((CH_PER_TILE, K), jnp.int32),
        pltpu.VMEM_SHARED((NPAD, DEGW), jnp.float32),
    ],
)(_sc_deg_body)


def _sc_edge_body(src_hbm, dst_hbm, h_hbm, out_hbm, src_v, dst_v, rows_v,
                  agg_sh, sem):
    c = lax.axis_index("c")
    s = lax.axis_index("s")
    _fill_rows(rows_v, K, D // 16, 0.0)
    for z in range(ROWS_PER_TILE // K):
        pltpu.sync_copy(rows_v, agg_sh.at[pl.ds(s * ROWS_PER_TILE + z * K, K)])
    plsc.subcore_barrier()
    base = c * CH_PER_CORE + s * CH_PER_TILE
    pltpu.sync_copy(src_hbm.at[pl.ds(base, CH_PER_TILE)], src_v)
    pltpu.sync_copy(dst_hbm.at[pl.ds(base, CH_PER_TILE)], dst_v)

    def body(j, _):
        pltpu.async_copy(h_hbm.at[src_v.at[j]], rows_v, sem).wait()
        pltpu.sync_copy(rows_v, agg_sh.at[dst_v.at[j]], add=True)
        return 0

    lax.fori_loop(0, CH_PER_TILE, body, 0)
    plsc.subcore_barrier()
    r0 = s * ROWS_PER_TILE
    pltpu.sync_copy(agg_sh.at[pl.ds(r0, ROWS_PER_TILE)],
                    out_hbm.at[c, pl.ds(r0, ROWS_PER_TILE)])


_sc_edge = functools.partial(
    pl.kernel,
    out_type=jax.ShapeDtypeStruct((NCORES, NPAD, D), jnp.float32),
    mesh=_mesh,
    scratch_types=[
        pltpu.VMEM((CH_PER_TILE, K), jnp.int32),
        pltpu.VMEM((CH_PER_TILE, K), jnp.int32),
        pltpu.VMEM((K, D), jnp.float32),
        pltpu.VMEM_SHARED((NPAD, D), jnp.float32),
        pltpu.SemaphoreType.DMA,
    ],
)(_sc_edge_body)


_BLK = 1280
_GRID = NPAD // _BLK


def _dis(d0, d1):
    return lax.rsqrt(d0[:, 0:1] + d1[:, 0:1] + 1.0)


def _tc_pre_body(d0, d1, x, w, o):
    o[...] = _dis(d0, d1) * jnp.dot(x[...], w[...],
                                    preferred_element_type=jnp.float32)


def _tc_mid_body(d0, d1, a0, a1, h, w, o):
    dis = _dis(d0, d1)
    y = jnp.maximum(dis * (a0[...] + a1[...] + h[...]), 0.0)
    o[...] = dis * jnp.dot(y, w[...], preferred_element_type=jnp.float32)


def _tc_post_body(d0, d1, a0, a1, h, o):
    o[...] = _dis(d0, d1) * (a0[...] + a1[...] + h[...])


_deg_spec = pl.BlockSpec((_BLK, DEGW), lambda i: (i, 0))
_row_spec = pl.BlockSpec((_BLK, D), lambda i: (i, 0))
_w_spec = pl.BlockSpec((D, D), lambda i: (0, 0))
_out_sds = jax.ShapeDtypeStruct((NPAD, D), jnp.float32)

_tc_pre = pl.pallas_call(
    _tc_pre_body, grid=(_GRID,),
    in_specs=[_deg_spec, _deg_spec, _row_spec, _w_spec],
    out_specs=_row_spec, out_shape=_out_sds)

_tc_mid = pl.pallas_call(
    _tc_mid_body, grid=(_GRID,),
    in_specs=[_deg_spec, _deg_spec, _row_spec, _row_spec, _row_spec, _w_spec],
    out_specs=_row_spec, out_shape=_out_sds)

_tc_post = pl.pallas_call(
    _tc_post_body, grid=(_GRID,),
    in_specs=[_deg_spec, _deg_spec, _row_spec, _row_spec, _row_spec],
    out_specs=_row_spec, out_shape=_out_sds)


def kernel(x, edge_index, W0, W1, W2):
    src = edge_index[0]
    dst = edge_index[1]
    src2 = jnp.concatenate(
        [src, jnp.full((E_PAD - E,), JUNK_SRC, jnp.int32)]).reshape(CHUNKS, K)
    dst2 = jnp.concatenate(
        [dst, jnp.full((E_PAD - E,), JUNK_DST, jnp.int32)]).reshape(CHUNKS, K)
    xp = jnp.pad(x, ((0, NPAD - N), (0, 0)))

    degp = _sc_deg(dst2)
    d0, d1 = degp[0], degp[1]
    h = _tc_pre(d0, d1, xp, W0)
    a = _sc_edge(src2, dst2, h)
    h = _tc_mid(d0, d1, a[0], a[1], h, W1)
    a = _sc_edge(src2, dst2, h)
    h = _tc_mid(d0, d1, a[0], a[1], h, W2)
    a = _sc_edge(src2, dst2, h)
    o = _tc_post(d0, d1, a[0], a[1], h)
    return o[:N]


# speedup vs baseline: 6.9975x; 6.9975x over previous
import functools

import jax
import jax.numpy as jnp
from jax import lax
from jax.experimental import pallas as pl
from jax.experimental.pallas import tpu as pltpu
from jax.experimental.pallas import tpu_sc as plsc

N = 10000
D = 128
E = 320000

NPAD = 10240
K = 128
CHUNKS = 2560
E_PAD = CHUNKS * K
NCORES = 2
NSUB = 16
CH_PER_CORE = CHUNKS // NCORES
CH_PER_TILE = CH_PER_CORE // NSUB
ROWS_PER_TILE = NPAD // NSUB
JUNK_SRC = N + 100
JUNK_DST = N + 200
DEGW = 16

_mesh = plsc.VectorSubcoreMesh(core_axis_name="c", subcore_axis_name="s")


def _fill_rows(ref, nrows, ncolchunks, value):
    v = jnp.full((16,), value, jnp.float32)

    def outer(r, _):
        def inner(k, _):
            ref[r, pl.ds(k * 16, 16)] = v
            return 0
        return lax.fori_loop(0, ncolchunks, inner, 0)

    lax.fori_loop(0, nrows, outer, 0)


def _sc_deg_body(dst_hbm, out_hbm, idx_v, tab_v, buf_v, res_v, deg_sh):
    c = lax.axis_index("c")
    s = lax.axis_index("s")
    zero16 = jnp.zeros((16,), jnp.float32)
    one16 = jnp.ones((16,), jnp.float32)

    def z(i, _):
        tab_v[pl.ds(i * 16, 16)] = zero16
        return 0

    lax.fori_loop(0, NPAD // 16, z, 0)
    base = c * CH_PER_CORE + s * CH_PER_TILE
    pltpu.sync_copy(dst_hbm.at[pl.ds(base, CH_PER_TILE)], idx_v)

    def body(j, _):
        def inner(k, _):
            idx = idx_v[j, pl.ds(k * 16, 16)]
            plsc.addupdate_scatter(tab_v, [idx], one16)
            return 0
        return lax.fori_loop(0, K // 16, inner, 0)

    lax.fori_loop(0, CH_PER_TILE, body, 0)
    pltpu.sync_copy(tab_v, deg_sh.at[pl.ds(s * NPAD, NPAD)])
    plsc.subcore_barrier()
    col0 = s * ROWS_PER_TILE

    def z2(i, _):
        res_v[pl.ds(i * 16, 16)] = zero16
        return 0

    lax.fori_loop(0, ROWS_PER_TILE // 16, z2, 0)
    for t in range(NSUB):
        pltpu.sync_copy(deg_sh.at[pl.ds(t * NPAD + col0, ROWS_PER_TILE)], buf_v)

        def acc(m, _):
            res_v[pl.ds(m * 16, 16)] = (res_v[pl.ds(m * 16, 16)]
                                        + buf_v[pl.ds(m * 16, 16)])
            return 0

        lax.fori_loop(0, ROWS_PER_TILE // 16, acc, 0)
    pltpu.sync_copy(res_v, out_hbm.at[pl.ds(c * NPAD + col0, ROWS_PER_TILE)])


_sc_deg = functools.partial(
    pl.kernel,
    out_type=jax.ShapeDtypeStruct((NCORES * NPAD,), jnp.float32),
    mesh=_mesh,
    scratch_types=[
        pltpu.VMEM((CH_PER_TILE, K), jnp.int32),
        pltpu.VMEM((NPAD,), jnp.float32),
        pltpu.VMEM((ROWS_PER_TILE,), jnp.float32),
        pltpu.VMEM((ROWS_PER_TILE,), jnp.float32),
        pltpu.VMEM_SHARED((NSUB * NPAD,), jnp.float32),
    ],
    compiler_params=pltpu.CompilerParams(needs_layout_passes=False),
)(_sc_deg_body)


def _sc_edge_body(src_hbm, dst_hbm, h_hbm, out_hbm, src_v, dst_v, rows_v,
                  agg_sh, sem):
    c = lax.axis_index("c")
    s = lax.axis_index("s")
    _fill_rows(rows_v, K, D // 16, 0.0)
    for z in range(ROWS_PER_TILE // K):
        pltpu.sync_copy(rows_v, agg_sh.at[pl.ds(s * ROWS_PER_TILE + z * K, K)])
    plsc.subcore_barrier()
    base = c * CH_PER_CORE + s * CH_PER_TILE
    pltpu.sync_copy(src_hbm.at[pl.ds(base, CH_PER_TILE)], src_v)
    pltpu.sync_copy(dst_hbm.at[pl.ds(base, CH_PER_TILE)], dst_v)

    def body(j, _):
        pltpu.async_copy(h_hbm.at[src_v.at[j]], rows_v, sem).wait()
        pltpu.sync_copy(rows_v, agg_sh.at[dst_v.at[j]], add=True)
        return 0

    lax.fori_loop(0, CH_PER_TILE, body, 0)
    plsc.subcore_barrier()
    r0 = s * ROWS_PER_TILE
    pltpu.sync_copy(agg_sh.at[pl.ds(r0, ROWS_PER_TILE)],
                    out_hbm.at[c, pl.ds(r0, ROWS_PER_TILE)])


_sc_edge = functools.partial(
    pl.kernel,
    out_type=jax.ShapeDtypeStruct((NCORES, NPAD, D), jnp.float32),
    mesh=_mesh,
    scratch_types=[
        pltpu.VMEM((CH_PER_TILE, K), jnp.int32),
        pltpu.VMEM((CH_PER_TILE, K), jnp.int32),
        pltpu.VMEM((K, D), jnp.float32),
        pltpu.VMEM_SHARED((NPAD, D), jnp.float32),
        pltpu.SemaphoreType.DMA,
    ],
)(_sc_edge_body)


_BLK = 1280
_GRID = NPAD // _BLK


def _dis(d0, d1):
    return lax.rsqrt(d0[...] + d1[...] + 1.0)


def _tc_pre_body(d0, d1, x, w, o):
    o[...] = _dis(d0, d1) * jnp.dot(x[...], w[...],
                                    preferred_element_type=jnp.float32)


def _tc_mid_body(d0, d1, a0, a1, h, w, o):
    dis = _dis(d0, d1)
    y = jnp.maximum(dis * (a0[...] + a1[...] + h[...]), 0.0)
    o[...] = dis * jnp.dot(y, w[...], preferred_element_type=jnp.float32)


def _tc_post_body(d0, d1, a0, a1, h, o):
    o[...] = _dis(d0, d1) * (a0[...] + a1[...] + h[...])


_deg_spec = pl.BlockSpec((_BLK, 1), lambda i: (i, 0))
_row_spec = pl.BlockSpec((_BLK, D), lambda i: (i, 0))
_w_spec = pl.BlockSpec((D, D), lambda i: (0, 0))
_out_sds = jax.ShapeDtypeStruct((NPAD, D), jnp.float32)

_tc_pre = pl.pallas_call(
    _tc_pre_body, grid=(_GRID,),
    in_specs=[_deg_spec, _deg_spec, _row_spec, _w_spec],
    out_specs=_row_spec, out_shape=_out_sds)

_tc_mid = pl.pallas_call(
    _tc_mid_body, grid=(_GRID,),
    in_specs=[_deg_spec, _deg_spec, _row_spec, _row_spec, _row_spec, _w_spec],
    out_specs=_row_spec, out_shape=_out_sds)

_tc_post = pl.pallas_call(
    _tc_post_body, grid=(_GRID,),
    in_specs=[_deg_spec, _deg_spec, _row_spec, _row_spec, _row_spec],
    out_specs=_row_spec, out_shape=_out_sds)


def kernel(x, edge_index, W0, W1, W2):
    src = edge_index[0]
    dst = edge_index[1]
    src2 = jnp.concatenate(
        [src, jnp.full((E_PAD - E,), JUNK_SRC, jnp.int32)]).reshape(CHUNKS, K)
    dst2 = jnp.concatenate(
        [dst, jnp.full((E_PAD - E,), JUNK_DST, jnp.int32)]).reshape(CHUNKS, K)
    xp = jnp.pad(x, ((0, NPAD - N), (0, 0)))

    degp = _sc_deg(dst2).reshape(NCORES, NPAD, 1)
    d0, d1 = degp[0], degp[1]
    h = _tc_pre(d0, d1, xp, W0)
    a = _sc_edge(src2, dst2, h)
    h = _tc_mid(d0, d1, a[0], a[1], h, W1)
    a = _sc_edge(src2, dst2, h)
    h = _tc_mid(d0, d1, a[0], a[1], h, W2)
    a = _sc_edge(src2, dst2, h)
    o = _tc_post(d0, d1, a[0], a[1], h)
    return o[:N]


# speedup vs baseline: 7.6203x; 1.0890x over previous
import functools

import jax
import jax.numpy as jnp
from jax import lax
from jax.experimental import pallas as pl
from jax.experimental.pallas import tpu as pltpu
from jax.experimental.pallas import tpu_sc as plsc

N = 10000
D = 128
E = 320000

NPAD = 10240
K = 128
CHUNKS = 2560
E_PAD = CHUNKS * K
NCORES = 2
NSUB = 16
CH_PER_CORE = CHUNKS // NCORES
CH_PER_TILE = CH_PER_CORE // NSUB
ROWS_PER_TILE = NPAD // NSUB
JUNK_SRC = N + 100
JUNK_DST = N + 200
DEGW = 16

_mesh = plsc.VectorSubcoreMesh(core_axis_name="c", subcore_axis_name="s")


def _fill_rows(ref, nrows, ncolchunks, value):
    v = jnp.full((16,), value, jnp.float32)

    def outer(r, _):
        def inner(k, _):
            ref[r, pl.ds(k * 16, 16)] = v
            return 0
        return lax.fori_loop(0, ncolchunks, inner, 0)

    lax.fori_loop(0, nrows, outer, 0)


def _sc_deg_body(dst_hbm, out_hbm, idx_v, tab_v, buf_v, res_v, deg_sh):
    c = lax.axis_index("c")
    s = lax.axis_index("s")
    zero16 = jnp.zeros((16,), jnp.float32)
    one16 = jnp.ones((16,), jnp.float32)

    def z(i, _):
        tab_v[pl.ds(i * 16, 16)] = zero16
        return 0

    lax.fori_loop(0, NPAD // 16, z, 0)
    base = c * CH_PER_CORE + s * CH_PER_TILE
    pltpu.sync_copy(dst_hbm.at[pl.ds(base, CH_PER_TILE)], idx_v)

    def body(j, _):
        def inner(k, _):
            idx = idx_v[j, pl.ds(k * 16, 16)]
            plsc.addupdate_scatter(tab_v, [idx], one16)
            return 0
        return lax.fori_loop(0, K // 16, inner, 0)

    lax.fori_loop(0, CH_PER_TILE, body, 0)
    pltpu.sync_copy(tab_v, deg_sh.at[pl.ds(s * NPAD, NPAD)])
    plsc.subcore_barrier()
    col0 = s * ROWS_PER_TILE

    def z2(i, _):
        res_v[pl.ds(i * 16, 16)] = zero16
        return 0

    lax.fori_loop(0, ROWS_PER_TILE // 16, z2, 0)
    for t in range(NSUB):
        pltpu.sync_copy(deg_sh.at[pl.ds(t * NPAD + col0, ROWS_PER_TILE)], buf_v)

        def acc(m, _):
            res_v[pl.ds(m * 16, 16)] = (res_v[pl.ds(m * 16, 16)]
                                        + buf_v[pl.ds(m * 16, 16)])
            return 0

        lax.fori_loop(0, ROWS_PER_TILE // 16, acc, 0)
    pltpu.sync_copy(res_v, out_hbm.at[pl.ds(c * NPAD + col0, ROWS_PER_TILE)])


_sc_deg = functools.partial(
    pl.kernel,
    out_type=jax.ShapeDtypeStruct((NCORES * NPAD,), jnp.float32),
    mesh=_mesh,
    scratch_types=[
        pltpu.VMEM((CH_PER_TILE, K), jnp.int32),
        pltpu.VMEM((NPAD,), jnp.float32),
        pltpu.VMEM((ROWS_PER_TILE,), jnp.float32),
        pltpu.VMEM((ROWS_PER_TILE,), jnp.float32),
        pltpu.VMEM_SHARED((NSUB * NPAD,), jnp.float32),
    ],
    compiler_params=pltpu.CompilerParams(needs_layout_passes=False),
)(_sc_deg_body)


NBUF = 2
PHASES = 2
CPP = CH_PER_TILE // PHASES
ROUNDS = CPP // NBUF


def _sc_edge_body(src_hbm, dst_hbm, h_hbm, out_hbm, src_v, dst_v,
                  b0, b1, agg_sh, g0, g1, s0, s1):
    rows = [b0, b1]
    gsem = [g0, g1]
    ssem = [s0, s1]
    c = lax.axis_index("c")
    s = lax.axis_index("s")
    _fill_rows(b0, K, D // 16, 0.0)
    for z in range(ROWS_PER_TILE // K):
        pltpu.sync_copy(b0, agg_sh.at[pl.ds(s * ROWS_PER_TILE + z * K, K)])
    plsc.subcore_barrier()
    base = c * CH_PER_CORE + s * CH_PER_TILE

    for p in range(PHASES):
        pltpu.sync_copy(src_hbm.at[pl.ds(base + p * CPP, CPP)], src_v)
        pltpu.sync_copy(dst_hbm.at[pl.ds(base + p * CPP, CPP)], dst_v)
        for b in range(NBUF):
            pltpu.async_copy(h_hbm.at[src_v.at[b]], rows[b], gsem[b])

        def rnd(r, _):
            for b in range(NBUF):
                j = r * NBUF + b
                pltpu.make_async_copy(h_hbm.at[src_v.at[0]], rows[b],
                                      gsem[b]).wait()
                pltpu.async_copy(rows[b], agg_sh.at[dst_v.at[j]], ssem[b],
                                 add=True)
            for b in range(NBUF):
                jn = r * NBUF + b + NBUF

                @pl.when(jn < CPP)
                def _(b=b, jn=jn):
                    pltpu.make_async_copy(rows[b], agg_sh.at[dst_v.at[0]],
                                          ssem[b]).wait()
                    pltpu.async_copy(h_hbm.at[src_v.at[jn]], rows[b], gsem[b])
            return 0

        lax.fori_loop(0, ROUNDS, rnd, 0)
        for b in range(NBUF):
            pltpu.make_async_copy(rows[b], agg_sh.at[dst_v.at[0]],
                                  ssem[b]).wait()
    plsc.subcore_barrier()
    r0 = s * ROWS_PER_TILE
    pltpu.sync_copy(agg_sh.at[pl.ds(r0, ROWS_PER_TILE)],
                    out_hbm.at[c, pl.ds(r0, ROWS_PER_TILE)])


_sc_edge = functools.partial(
    pl.kernel,
    out_type=jax.ShapeDtypeStruct((NCORES, NPAD, D), jnp.float32),
    mesh=_mesh,
    scratch_types=[
        pltpu.VMEM((CPP, K), jnp.int32),
        pltpu.VMEM((CPP, K), jnp.int32),
    ] + [pltpu.VMEM((K, D), jnp.float32)] * NBUF
      + [pltpu.VMEM_SHARED((NPAD, D), jnp.float32)]
      + [pltpu.SemaphoreType.DMA] * (2 * NBUF),
)(_sc_edge_body)


_BLK = 1280
_GRID = NPAD // _BLK


def _dis(d0, d1):
    return lax.rsqrt(d0[...] + d1[...] + 1.0)


def _tc_pre_body(d0, d1, x, w, o):
    o[...] = _dis(d0, d1) * jnp.dot(x[...], w[...],
                                    preferred_element_type=jnp.float32)


def _tc_mid_body(d0, d1, a0, a1, h, w, o):
    dis = _dis(d0, d1)
    y = jnp.maximum(dis * (a0[...] + a1[...] + h[...]), 0.0)
    o[...] = dis * jnp.dot(y, w[...], preferred_element_type=jnp.float32)


def _tc_post_body(d0, d1, a0, a1, h, o):
    o[...] = _dis(d0, d1) * (a0[...] + a1[...] + h[...])


_deg_spec = pl.BlockSpec((_BLK, 1), lambda i: (i, 0))
_row_spec = pl.BlockSpec((_BLK, D), lambda i: (i, 0))
_w_spec = pl.BlockSpec((D, D), lambda i: (0, 0))
_out_sds = jax.ShapeDtypeStruct((NPAD, D), jnp.float32)

_tc_pre = pl.pallas_call(
    _tc_pre_body, grid=(_GRID,),
    in_specs=[_deg_spec, _deg_spec, _row_spec, _w_spec],
    out_specs=_row_spec, out_shape=_out_sds)

_tc_mid = pl.pallas_call(
    _tc_mid_body, grid=(_GRID,),
    in_specs=[_deg_spec, _deg_spec, _row_spec, _row_spec, _row_spec, _w_spec],
    out_specs=_row_spec, out_shape=_out_sds)

_tc_post = pl.pallas_call(
    _tc_post_body, grid=(_GRID,),
    in_specs=[_deg_spec, _deg_spec, _row_spec, _row_spec, _row_spec],
    out_specs=_row_spec, out_shape=_out_sds)


def kernel(x, edge_index, W0, W1, W2):
    src = edge_index[0]
    dst = edge_index[1]
    src2 = jnp.concatenate(
        [src, jnp.full((E_PAD - E,), JUNK_SRC, jnp.int32)]).reshape(CHUNKS, K)
    dst2 = jnp.concatenate(
        [dst, jnp.full((E_PAD - E,), JUNK_DST, jnp.int32)]).reshape(CHUNKS, K)
    xp = jnp.pad(x, ((0, NPAD - N), (0, 0)))

    degp = _sc_deg(dst2).reshape(NCORES, NPAD, 1)
    d0, d1 = degp[0], degp[1]
    h = _tc_pre(d0, d1, xp, W0)
    a = _sc_edge(src2, dst2, h)
    h = _tc_mid(d0, d1, a[0], a[1], h, W1)
    a = _sc_edge(src2, dst2, h)
    h = _tc_mid(d0, d1, a[0], a[1], h, W2)
    a = _sc_edge(src2, dst2, h)
    o = _tc_post(d0, d1, a[0], a[1], h)
    return o[:N]


# speedup vs baseline: 21.5250x; 2.8247x over previous
import functools

import jax
import jax.numpy as jnp
from jax import lax
from jax.experimental import pallas as pl
from jax.experimental.pallas import tpu as pltpu
from jax.experimental.pallas import tpu_sc as plsc

N = 10000
D = 128
E = 320000

NPAD = 10240
K = 128
CHUNKS = 2560
E_PAD = CHUNKS * K
NCORES = 2
NSUB = 16
CH_PER_CORE = CHUNKS // NCORES
CH_PER_TILE = CH_PER_CORE // NSUB
ROWS_PER_TILE = NPAD // NSUB
JUNK_SRC0 = N
N_JSRC = 80
JUNK_DST0 = N + 80
N_JDST = 160
DEGW = 16

_mesh = plsc.VectorSubcoreMesh(core_axis_name="c", subcore_axis_name="s")


def _fill_rows(ref, nrows, ncolchunks, value):
    v = jnp.full((16,), value, jnp.float32)

    def outer(r, _):
        def inner(k, _):
            ref[r, pl.ds(k * 16, 16)] = v
            return 0
        return lax.fori_loop(0, ncolchunks, inner, 0)

    lax.fori_loop(0, nrows, outer, 0)


def _sc_deg_body(dst_hbm, out_hbm, idx_v, tab_v, buf_v, res_v, deg_sh):
    c = lax.axis_index("c")
    s = lax.axis_index("s")
    zero16 = jnp.zeros((16,), jnp.float32)
    one16 = jnp.ones((16,), jnp.float32)

    def z(i, _):
        tab_v[pl.ds(i * 16, 16)] = zero16
        return 0

    lax.fori_loop(0, NPAD // 16, z, 0)
    base = c * CH_PER_CORE + s * CH_PER_TILE
    pltpu.sync_copy(dst_hbm.at[pl.ds(base, CH_PER_TILE)], idx_v)

    def body(j, _):
        def inner(k, _):
            idx = idx_v[j, pl.ds(k * 16, 16)]
            plsc.addupdate_scatter(tab_v, [idx], one16)
            return 0
        return lax.fori_loop(0, K // 16, inner, 0)

    lax.fori_loop(0, CH_PER_TILE, body, 0)
    pltpu.sync_copy(tab_v, deg_sh.at[pl.ds(s * NPAD, NPAD)])
    plsc.subcore_barrier()
    col0 = s * ROWS_PER_TILE

    def z2(i, _):
        res_v[pl.ds(i * 16, 16)] = zero16
        return 0

    lax.fori_loop(0, ROWS_PER_TILE // 16, z2, 0)
    for t in range(NSUB):
        pltpu.sync_copy(deg_sh.at[pl.ds(t * NPAD + col0, ROWS_PER_TILE)], buf_v)

        def acc(m, _):
            res_v[pl.ds(m * 16, 16)] = (res_v[pl.ds(m * 16, 16)]
                                        + buf_v[pl.ds(m * 16, 16)])
            return 0

        lax.fori_loop(0, ROWS_PER_TILE // 16, acc, 0)
    pltpu.sync_copy(res_v, out_hbm.at[pl.ds(c * NPAD + col0, ROWS_PER_TILE)])


_sc_deg = functools.partial(
    pl.kernel,
    out_type=jax.ShapeDtypeStruct((NCORES * NPAD,), jnp.float32),
    mesh=_mesh,
    scratch_types=[
        pltpu.VMEM((CH_PER_TILE, K), jnp.int32),
        pltpu.VMEM((NPAD,), jnp.float32),
        pltpu.VMEM((ROWS_PER_TILE,), jnp.float32),
        pltpu.VMEM((ROWS_PER_TILE,), jnp.float32),
        pltpu.VMEM_SHARED((NSUB * NPAD,), jnp.float32),
    ],
    compiler_params=pltpu.CompilerParams(needs_layout_passes=False),
)(_sc_deg_body)


NBUF = 2
PHASES = 2
CPP = CH_PER_TILE // PHASES
ROUNDS = CPP // NBUF


def _sc_edge_body(src_hbm, dst_hbm, h_hbm, out_hbm, src_v, dst_v,
                  b0, b1, agg_sh, g0, g1, s0, s1):
    rows = [b0, b1]
    gsem = [g0, g1]
    ssem = [s0, s1]
    c = lax.axis_index("c")
    s = lax.axis_index("s")
    _fill_rows(b0, K, D // 16, 0.0)
    for z in range(ROWS_PER_TILE // K):
        pltpu.sync_copy(b0, agg_sh.at[pl.ds(s * ROWS_PER_TILE + z * K, K)])
    plsc.subcore_barrier()
    base = c * CH_PER_CORE + s * CH_PER_TILE

    for p in range(PHASES):
        pltpu.sync_copy(src_hbm.at[pl.ds(base + p * CPP, CPP)], src_v)
        pltpu.sync_copy(dst_hbm.at[pl.ds(base + p * CPP, CPP)], dst_v)
        for b in range(NBUF):
            pltpu.async_copy(h_hbm.at[src_v.at[b]], rows[b], gsem[b])

        def rnd(r, _):
            for b in range(NBUF):
                j = r * NBUF + b
                pltpu.make_async_copy(h_hbm.at[src_v.at[0]], rows[b],
                                      gsem[b]).wait()
                pltpu.async_copy(rows[b], agg_sh.at[dst_v.at[j]], ssem[b],
                                 add=True)
            for b in range(NBUF):
                jn = r * NBUF + b + NBUF

                @pl.when(jn < CPP)
                def _(b=b, jn=jn):
                    pltpu.make_async_copy(rows[b], agg_sh.at[dst_v.at[0]],
                                          ssem[b]).wait()
                    pltpu.async_copy(h_hbm.at[src_v.at[jn]], rows[b], gsem[b])
            return 0

        lax.fori_loop(0, ROUNDS, rnd, 0)
        for b in range(NBUF):
            pltpu.make_async_copy(rows[b], agg_sh.at[dst_v.at[0]],
                                  ssem[b]).wait()
    plsc.subcore_barrier()
    r0 = s * ROWS_PER_TILE
    pltpu.sync_copy(agg_sh.at[pl.ds(r0, ROWS_PER_TILE)],
                    out_hbm.at[c, pl.ds(r0, ROWS_PER_TILE)])


_sc_edge = functools.partial(
    pl.kernel,
    out_type=jax.ShapeDtypeStruct((NCORES, NPAD, D), jnp.float32),
    mesh=_mesh,
    scratch_types=[
        pltpu.VMEM((CPP, K), jnp.int32),
        pltpu.VMEM((CPP, K), jnp.int32),
    ] + [pltpu.VMEM((K, D), jnp.float32)] * NBUF
      + [pltpu.VMEM_SHARED((NPAD, D), jnp.float32)]
      + [pltpu.SemaphoreType.DMA] * (2 * NBUF),
)(_sc_edge_body)


_BLK = 1280
_GRID = NPAD // _BLK


def _dis(d0, d1):
    return lax.rsqrt(d0[...] + d1[...] + 1.0)


def _tc_pre_body(d0, d1, x, w, o):
    o[...] = _dis(d0, d1) * jnp.dot(x[...], w[...],
                                    preferred_element_type=jnp.float32)


def _tc_mid_body(d0, d1, a0, a1, h, w, o):
    dis = _dis(d0, d1)
    y = jnp.maximum(dis * (a0[...] + a1[...] + h[...]), 0.0)
    o[...] = dis * jnp.dot(y, w[...], preferred_element_type=jnp.float32)


def _tc_post_body(d0, d1, a0, a1, h, o):
    o[...] = _dis(d0, d1) * (a0[...] + a1[...] + h[...])


_deg_spec = pl.BlockSpec((_BLK, 1), lambda i: (i, 0))
_row_spec = pl.BlockSpec((_BLK, D), lambda i: (i, 0))
_w_spec = pl.BlockSpec((D, D), lambda i: (0, 0))
_out_sds = jax.ShapeDtypeStruct((NPAD, D), jnp.float32)

_tc_pre = pl.pallas_call(
    _tc_pre_body, grid=(_GRID,),
    in_specs=[_deg_spec, _deg_spec, _row_spec, _w_spec],
    out_specs=_row_spec, out_shape=_out_sds)

_tc_mid = pl.pallas_call(
    _tc_mid_body, grid=(_GRID,),
    in_specs=[_deg_spec, _deg_spec, _row_spec, _row_spec, _row_spec, _w_spec],
    out_specs=_row_spec, out_shape=_out_sds)

_tc_post = pl.pallas_call(
    _tc_post_body, grid=(_GRID,),
    in_specs=[_deg_spec, _deg_spec, _row_spec, _row_spec, _row_spec],
    out_specs=_row_spec, out_shape=_out_sds)


def kernel(x, edge_index, W0, W1, W2):
    src = edge_index[0]
    dst = edge_index[1]
    pad_i = jnp.arange(E_PAD - E, dtype=jnp.int32)
    src2 = jnp.concatenate(
        [src, JUNK_SRC0 + pad_i % N_JSRC]).reshape(CHUNKS, K)
    dst2 = jnp.concatenate(
        [dst, JUNK_DST0 + pad_i % N_JDST]).reshape(CHUNKS, K)
    xp = jnp.pad(x, ((0, NPAD - N), (0, 0)))

    degp = _sc_deg(dst2).reshape(NCORES, NPAD, 1)
    d0, d1 = degp[0], degp[1]
    h = _tc_pre(d0, d1, xp, W0)
    a = _sc_edge(src2, dst2, h)
    h = _tc_mid(d0, d1, a[0], a[1], h, W1)
    a = _sc_edge(src2, dst2, h)
    h = _tc_mid(d0, d1, a[0], a[1], h, W2)
    a = _sc_edge(src2, dst2, h)
    o = _tc_post(d0, d1, a[0], a[1], h)
    return o[:N]


# speedup vs baseline: 22.3912x; 1.0402x over previous
import functools

import jax
import jax.numpy as jnp
from jax import lax
from jax.experimental import pallas as pl
from jax.experimental.pallas import tpu as pltpu
from jax.experimental.pallas import tpu_sc as plsc

N = 10000
D = 128
E = 320000

NPAD = 10240
K = 128
CHUNKS = 2560
E_PAD = CHUNKS * K
NCORES = 2
NSUB = 16
CH_PER_CORE = CHUNKS // NCORES
CH_PER_TILE = CH_PER_CORE // NSUB
ROWS_PER_TILE = NPAD // NSUB
JUNK_SRC0 = N
N_JSRC = 80
JUNK_DST0 = N + 80
N_JDST = 160
DEGW = 16

_mesh = plsc.VectorSubcoreMesh(core_axis_name="c", subcore_axis_name="s")


def _fill_rows(ref, nrows, ncolchunks, value):
    v = jnp.full((16,), value, jnp.float32)

    def outer(r, _):
        for k in range(ncolchunks):
            ref[r, pl.ds(k * 16, 16)] = v
        return 0

    lax.fori_loop(0, nrows, outer, 0)


def _sc_deg_body(dst_hbm, out_hbm, idx_v, tab_v, buf_v, res_v, deg_sh):
    c = lax.axis_index("c")
    s = lax.axis_index("s")
    zero16 = jnp.zeros((16,), jnp.float32)
    one16 = jnp.ones((16,), jnp.float32)

    def z(i, _):
        tab_v[pl.ds(i * 16, 16)] = zero16
        return 0

    lax.fori_loop(0, NPAD // 16, z, 0)
    base = c * CH_PER_CORE + s * CH_PER_TILE
    pltpu.sync_copy(dst_hbm.at[pl.ds(base, CH_PER_TILE)], idx_v)

    def body(j, _):
        def inner(k, _):
            idx = idx_v[j, pl.ds(k * 16, 16)]
            plsc.addupdate_scatter(tab_v, [idx], one16)
            return 0
        return lax.fori_loop(0, K // 16, inner, 0)

    lax.fori_loop(0, CH_PER_TILE, body, 0)
    pltpu.sync_copy(tab_v, deg_sh.at[pl.ds(s * NPAD, NPAD)])
    plsc.subcore_barrier()
    col0 = s * ROWS_PER_TILE

    def z2(i, _):
        res_v[pl.ds(i * 16, 16)] = zero16
        return 0

    lax.fori_loop(0, ROWS_PER_TILE // 16, z2, 0)
    for t in range(NSUB):
        pltpu.sync_copy(deg_sh.at[pl.ds(t * NPAD + col0, ROWS_PER_TILE)], buf_v)

        def acc(m, _):
            res_v[pl.ds(m * 16, 16)] = (res_v[pl.ds(m * 16, 16)]
                                        + buf_v[pl.ds(m * 16, 16)])
            return 0

        lax.fori_loop(0, ROWS_PER_TILE // 16, acc, 0)
    pltpu.sync_copy(res_v, out_hbm.at[pl.ds(c * NPAD + col0, ROWS_PER_TILE)])


_sc_deg = functools.partial(
    pl.kernel,
    out_type=jax.ShapeDtypeStruct((NCORES * NPAD,), jnp.float32),
    mesh=_mesh,
    scratch_types=[
        pltpu.VMEM((CH_PER_TILE, K), jnp.int32),
        pltpu.VMEM((NPAD,), jnp.float32),
        pltpu.VMEM((ROWS_PER_TILE,), jnp.float32),
        pltpu.VMEM((ROWS_PER_TILE,), jnp.float32),
        pltpu.VMEM_SHARED((NSUB * NPAD,), jnp.float32),
    ],
    compiler_params=pltpu.CompilerParams(needs_layout_passes=False),
)(_sc_deg_body)


NBUF = 2
PHASES = 2
CPP = CH_PER_TILE // PHASES
ROUNDS = CPP // NBUF


def _sc_edge_body(src_hbm, dst_hbm, h_hbm, out_hbm, src_v, dst_v,
                  b0, b1, agg_sh, g0, g1, s0, s1):
    rows = [b0, b1]
    gsem = [g0, g1]
    ssem = [s0, s1]
    c = lax.axis_index("c")
    s = lax.axis_index("s")
    _fill_rows(b0, K, D // 16, 0.0)
    for z in range(ROWS_PER_TILE // K):
        pltpu.sync_copy(b0, agg_sh.at[pl.ds(s * ROWS_PER_TILE + z * K, K)])
    plsc.subcore_barrier()
    base = c * CH_PER_CORE + s * CH_PER_TILE

    for p in range(PHASES):
        pltpu.sync_copy(src_hbm.at[pl.ds(base + p * CPP, CPP)], src_v)
        pltpu.sync_copy(dst_hbm.at[pl.ds(base + p * CPP, CPP)], dst_v)
        for b in range(NBUF):
            pltpu.async_copy(h_hbm.at[src_v.at[b]], rows[b], gsem[b])

        def rnd(r, _):
            for b in range(NBUF):
                j = r * NBUF + b
                pltpu.make_async_copy(h_hbm.at[src_v.at[0]], rows[b],
                                      gsem[b]).wait()
                pltpu.async_copy(rows[b], agg_sh.at[dst_v.at[j]], ssem[b],
                                 add=True)
            for b in range(NBUF):
                jn = r * NBUF + b + NBUF

                @pl.when(jn < CPP)
                def _(b=b, jn=jn):
                    pltpu.make_async_copy(rows[b], agg_sh.at[dst_v.at[0]],
                                          ssem[b]).wait()
                    pltpu.async_copy(h_hbm.at[src_v.at[jn]], rows[b], gsem[b])
            return 0

        lax.fori_loop(0, ROUNDS, rnd, 0)
        for b in range(NBUF):
            pltpu.make_async_copy(rows[b], agg_sh.at[dst_v.at[0]],
                                  ssem[b]).wait()
    plsc.subcore_barrier()
    r0 = s * ROWS_PER_TILE
    pltpu.sync_copy(agg_sh.at[pl.ds(r0, ROWS_PER_TILE)],
                    out_hbm.at[c, pl.ds(r0, ROWS_PER_TILE)])


_sc_edge = functools.partial(
    pl.kernel,
    out_type=jax.ShapeDtypeStruct((NCORES, NPAD, D), jnp.float32),
    mesh=_mesh,
    scratch_types=[
        pltpu.VMEM((CPP, K), jnp.int32),
        pltpu.VMEM((CPP, K), jnp.int32),
    ] + [pltpu.VMEM((K, D), jnp.float32)] * NBUF
      + [pltpu.VMEM_SHARED((NPAD, D), jnp.float32)]
      + [pltpu.SemaphoreType.DMA] * (2 * NBUF),
)(_sc_edge_body)


_BLK = 1000
_GRID = N // _BLK


def _dis(d0, d1):
    return lax.rsqrt(d0[0] + d1[0] + 1.0)


def _tc_pre_body(d0, d1, x, w, o):
    o[...] = _dis(d0, d1) * jnp.dot(x[...], w[...],
                                    preferred_element_type=jnp.float32)


def _tc_mid_body(d0, d1, a0, a1, h, w, o):
    dis = _dis(d0, d1)
    y = jnp.maximum(dis * (a0[0] + a1[0] + h[...]), 0.0)
    o[...] = dis * jnp.dot(y, w[...], preferred_element_type=jnp.float32)


def _tc_post_body(d0, d1, a0, a1, h, o):
    o[...] = _dis(d0, d1) * (a0[0] + a1[0] + h[...])


_d0_spec = pl.BlockSpec((1, _BLK, 1), lambda i: (0, i, 0))
_d1_spec = pl.BlockSpec((1, _BLK, 1), lambda i: (1, i, 0))
_a0_spec = pl.BlockSpec((1, _BLK, D), lambda i: (0, i, 0))
_a1_spec = pl.BlockSpec((1, _BLK, D), lambda i: (1, i, 0))
_row_spec = pl.BlockSpec((_BLK, D), lambda i: (i, 0))
_w_spec = pl.BlockSpec((D, D), lambda i: (0, 0))
_out_sds = jax.ShapeDtypeStruct((NPAD, D), jnp.float32)

_tc_pre = pl.pallas_call(
    _tc_pre_body, grid=(_GRID,),
    in_specs=[_d0_spec, _d1_spec, _row_spec, _w_spec],
    out_specs=_row_spec, out_shape=_out_sds)

_tc_mid = pl.pallas_call(
    _tc_mid_body, grid=(_GRID,),
    in_specs=[_d0_spec, _d1_spec, _a0_spec, _a1_spec, _row_spec, _w_spec],
    out_specs=_row_spec, out_shape=_out_sds)

_tc_post = pl.pallas_call(
    _tc_post_body, grid=(_GRID,),
    in_specs=[_d0_spec, _d1_spec, _a0_spec, _a1_spec, _row_spec],
    out_specs=_row_spec, out_shape=jax.ShapeDtypeStruct((N, D), jnp.float32))


def kernel(x, edge_index, W0, W1, W2):
    src = edge_index[0]
    dst = edge_index[1]
    pad_i = jnp.arange(E_PAD - E, dtype=jnp.int32)
    src2 = jnp.concatenate(
        [src, JUNK_SRC0 + pad_i % N_JSRC]).reshape(CHUNKS, K)
    dst2 = jnp.concatenate(
        [dst, JUNK_DST0 + pad_i % N_JDST]).reshape(CHUNKS, K)

    degp = _sc_deg(dst2).reshape(NCORES, NPAD, 1)
    h = _tc_pre(degp, degp, x, W0)
    a = _sc_edge(src2, dst2, h)
    h = _tc_mid(degp, degp, a, a, h, W1)
    a = _sc_edge(src2, dst2, h)
    h = _tc_mid(degp, degp, a, a, h, W2)
    a = _sc_edge(src2, dst2, h)
    return _tc_post(degp, degp, a, a, h)


# speedup vs baseline: 22.5402x; 1.0067x over previous
import functools

import jax
import jax.numpy as jnp
from jax import lax
from jax.experimental import pallas as pl
from jax.experimental.pallas import tpu as pltpu
from jax.experimental.pallas import tpu_sc as plsc

N = 10000
D = 128
E = 320000

NPAD = 10240
K = 128
CHUNKS = 2560
E_PAD = CHUNKS * K
NCORES = 2
NSUB = 16
CH_PER_CORE = CHUNKS // NCORES
CH_PER_TILE = CH_PER_CORE // NSUB
ROWS_PER_TILE = NPAD // NSUB
JUNK_SRC0 = N
N_JSRC = 80
JUNK_DST0 = N + 80
N_JDST = 160
DEGW = 16

_mesh = plsc.VectorSubcoreMesh(core_axis_name="c", subcore_axis_name="s")


def _fill_rows(ref, nrows, ncolchunks, value):
    v = jnp.full((16,), value, jnp.float32)

    def outer(r, _):
        for k in range(ncolchunks):
            ref[r, pl.ds(k * 16, 16)] = v
        return 0

    lax.fori_loop(0, nrows, outer, 0)


def _sc_deg_body(dst_hbm, out_hbm, idx_v, tab_v, buf_v, res_v, deg_sh):
    c = lax.axis_index("c")
    s = lax.axis_index("s")
    zero16 = jnp.zeros((16,), jnp.float32)
    one16 = jnp.ones((16,), jnp.float32)

    def z(i, _):
        for k in range(16):
            tab_v[pl.ds(i * 256 + k * 16, 16)] = zero16
        return 0

    lax.fori_loop(0, NPAD // 256, z, 0)
    base = c * CH_PER_CORE + s * CH_PER_TILE
    pltpu.sync_copy(dst_hbm.at[pl.ds(base, CH_PER_TILE)], idx_v)

    def body(j, _):
        for k in range(K // 16):
            idx = idx_v[j, pl.ds(k * 16, 16)]
            plsc.addupdate_scatter(tab_v, [idx], one16)
        return 0

    lax.fori_loop(0, CH_PER_TILE, body, 0)
    pltpu.sync_copy(tab_v, deg_sh.at[pl.ds(s * NPAD, NPAD)])
    plsc.subcore_barrier()
    col0 = s * ROWS_PER_TILE
    for k in range(ROWS_PER_TILE // 16):
        res_v[pl.ds(k * 16, 16)] = zero16
    for t in range(NSUB):
        pltpu.sync_copy(deg_sh.at[pl.ds(t * NPAD + col0, ROWS_PER_TILE)], buf_v)

        def acc(m, _):
            for k in range(8):
                i0 = m * 128 + k * 16
                res_v[pl.ds(i0, 16)] = (res_v[pl.ds(i0, 16)]
                                        + buf_v[pl.ds(i0, 16)])
            return 0

        lax.fori_loop(0, ROWS_PER_TILE // 128, acc, 0)
    pltpu.sync_copy(res_v, out_hbm.at[pl.ds(c * NPAD + col0, ROWS_PER_TILE)])


_sc_deg = functools.partial(
    pl.kernel,
    out_type=jax.ShapeDtypeStruct((NCORES * NPAD,), jnp.float32),
    mesh=_mesh,
    scratch_types=[
        pltpu.VMEM((CH_PER_TILE, K), jnp.int32),
        pltpu.VMEM((NPAD,), jnp.float32),
        pltpu.VMEM((ROWS_PER_TILE,), jnp.float32),
        pltpu.VMEM((ROWS_PER_TILE,), jnp.float32),
        pltpu.VMEM_SHARED((NSUB * NPAD,), jnp.float32),
    ],
    compiler_params=pltpu.CompilerParams(needs_layout_passes=False),
)(_sc_deg_body)


NBUF = 2
PHASES = 2
CPP = CH_PER_TILE // PHASES
ROUNDS = CPP // NBUF


def _sc_edge_body(src_hbm, dst_hbm, h_hbm, out_hbm, src_v, dst_v,
                  b0, b1, agg_sh, g0, g1, s0, s1):
    rows = [b0, b1]
    gsem = [g0, g1]
    ssem = [s0, s1]
    c = lax.axis_index("c")
    s = lax.axis_index("s")
    _fill_rows(b0, K, D // 16, 0.0)
    for z in range(ROWS_PER_TILE // K):
        pltpu.sync_copy(b0, agg_sh.at[pl.ds(s * ROWS_PER_TILE + z * K, K)])
    plsc.subcore_barrier()
    base = c * CH_PER_CORE + s * CH_PER_TILE

    def phase(p, _):
        pltpu.sync_copy(src_hbm.at[pl.ds(base + p * CPP, CPP)], src_v)
        pltpu.sync_copy(dst_hbm.at[pl.ds(base + p * CPP, CPP)], dst_v)
        for b in range(NBUF):
            pltpu.async_copy(h_hbm.at[src_v.at[b]], rows[b], gsem[b])

        def rnd(r, _):
            for b in range(NBUF):
                j = r * NBUF + b
                pltpu.make_async_copy(h_hbm.at[src_v.at[0]], rows[b],
                                      gsem[b]).wait()
                pltpu.async_copy(rows[b], agg_sh.at[dst_v.at[j]], ssem[b],
                                 add=True)
            for b in range(NBUF):
                jn = r * NBUF + b + NBUF

                @pl.when(jn < CPP)
                def _(b=b, jn=jn):
                    pltpu.make_async_copy(rows[b], agg_sh.at[dst_v.at[0]],
                                          ssem[b]).wait()
                    pltpu.async_copy(h_hbm.at[src_v.at[jn]], rows[b], gsem[b])
            return 0

        lax.fori_loop(0, ROUNDS, rnd, 0)
        for b in range(NBUF):
            pltpu.make_async_copy(rows[b], agg_sh.at[dst_v.at[0]],
                                  ssem[b]).wait()
        return 0

    lax.fori_loop(0, PHASES, phase, 0)
    plsc.subcore_barrier()
    r0 = s * ROWS_PER_TILE
    pltpu.sync_copy(agg_sh.at[pl.ds(r0, ROWS_PER_TILE)],
                    out_hbm.at[c, pl.ds(r0, ROWS_PER_TILE)])


_sc_edge = functools.partial(
    pl.kernel,
    out_type=jax.ShapeDtypeStruct((NCORES, NPAD, D), jnp.float32),
    mesh=_mesh,
    scratch_types=[
        pltpu.VMEM((CPP, K), jnp.int32),
        pltpu.VMEM((CPP, K), jnp.int32),
    ] + [pltpu.VMEM((K, D), jnp.float32)] * NBUF
      + [pltpu.VMEM_SHARED((NPAD, D), jnp.float32)]
      + [pltpu.SemaphoreType.DMA] * (2 * NBUF),
)(_sc_edge_body)


_BLK = 1000
_GRID = N // _BLK


def _dis(d0, d1):
    return lax.rsqrt(d0[0] + d1[0] + 1.0)


def _tc_pre_body(d0, d1, x, w, o):
    o[...] = _dis(d0, d1) * jnp.dot(x[...], w[...],
                                    preferred_element_type=jnp.float32)


def _tc_mid_body(d0, d1, a0, a1, h, w, o):
    dis = _dis(d0, d1)
    y = jnp.maximum(dis * (a0[0] + a1[0] + h[...]), 0.0)
    o[...] = dis * jnp.dot(y, w[...], preferred_element_type=jnp.float32)


def _tc_post_body(d0, d1, a0, a1, h, o):
    o[...] = _dis(d0, d1) * (a0[0] + a1[0] + h[...])


_d0_spec = pl.BlockSpec((1, _BLK, 1), lambda i: (0, i, 0))
_d1_spec = pl.BlockSpec((1, _BLK, 1), lambda i: (1, i, 0))
_a0_spec = pl.BlockSpec((1, _BLK, D), lambda i: (0, i, 0))
_a1_spec = pl.BlockSpec((1, _BLK, D), lambda i: (1, i, 0))
_row_spec = pl.BlockSpec((_BLK, D), lambda i: (i, 0))
_w_spec = pl.BlockSpec((D, D), lambda i: (0, 0))
_out_sds = jax.ShapeDtypeStruct((NPAD, D), jnp.float32)

_tc_pre = pl.pallas_call(
    _tc_pre_body, grid=(_GRID,),
    in_specs=[_d0_spec, _d1_spec, _row_spec, _w_spec],
    out_specs=_row_spec, out_shape=_out_sds)

_tc_mid = pl.pallas_call(
    _tc_mid_body, grid=(_GRID,),
    in_specs=[_d0_spec, _d1_spec, _a0_spec, _a1_spec, _row_spec, _w_spec],
    out_specs=_row_spec, out_shape=_out_sds)

_tc_post = pl.pallas_call(
    _tc_post_body, grid=(_GRID,),
    in_specs=[_d0_spec, _d1_spec, _a0_spec, _a1_spec, _row_spec],
    out_specs=_row_spec, out_shape=jax.ShapeDtypeStruct((N, D), jnp.float32))


def kernel(x, edge_index, W0, W1, W2):
    src = edge_index[0]
    dst = edge_index[1]
    pad_i = jnp.arange(E_PAD - E, dtype=jnp.int32)
    src2 = jnp.concatenate(
        [src, JUNK_SRC0 + pad_i % N_JSRC]).reshape(CHUNKS, K)
    dst2 = jnp.concatenate(
        [dst, JUNK_DST0 + pad_i % N_JDST]).reshape(CHUNKS, K)

    degp = _sc_deg(dst2).reshape(NCORES, NPAD, 1)
    h = _tc_pre(degp, degp, x, W0)
    a = _sc_edge(src2, dst2, h)
    h = _tc_mid(degp, degp, a, a, h, W1)
    a = _sc_edge(src2, dst2, h)
    h = _tc_mid(degp, degp, a, a, h, W2)
    a = _sc_edge(src2, dst2, h)
    return _tc_post(degp, degp, a, a, h)


# speedup vs baseline: 22.7480x; 1.0092x over previous
import functools

import jax
import jax.numpy as jnp
from jax import lax
from jax.experimental import pallas as pl
from jax.experimental.pallas import tpu as pltpu
from jax.experimental.pallas import tpu_sc as plsc

N = 10000
D = 128
E = 320000

NPAD = 10240
K = 125
CHUNKS = E // K
NCORES = 2
NSUB = 16
CH_PER_CORE = CHUNKS // NCORES
CH_PER_TILE = CH_PER_CORE // NSUB
ROWS_PER_TILE = NPAD // NSUB

_mesh = plsc.VectorSubcoreMesh(core_axis_name="c", subcore_axis_name="s")


def _fill_rows(ref, nrows, ncolchunks, value):
    v = jnp.full((16,), value, jnp.float32)

    def outer(r, _):
        for k in range(ncolchunks):
            ref[r, pl.ds(k * 16, 16)] = v
        return 0

    lax.fori_loop(0, nrows, outer, 0)


def _sc_deg_body(dst_hbm, out_hbm, idx_v, tab_v, buf_v, res_v, deg_sh):
    c = lax.axis_index("c")
    s = lax.axis_index("s")
    zero16 = jnp.zeros((16,), jnp.float32)
    one16 = jnp.ones((16,), jnp.float32)

    def z(i, _):
        for k in range(16):
            tab_v[pl.ds(i * 256 + k * 16, 16)] = zero16
        return 0

    lax.fori_loop(0, NPAD // 256, z, 0)
    base = c * CH_PER_CORE + s * CH_PER_TILE
    pltpu.sync_copy(dst_hbm.at[pl.ds(base, CH_PER_TILE)], idx_v)

    tail_mask = lax.iota(jnp.int32, 16) >= (16 - (K - (K // 16) * 16))

    def body(j, _):
        for k in range(K // 16):
            idx = idx_v[j, pl.ds(k * 16, 16)]
            plsc.addupdate_scatter(tab_v, [idx], one16)
        idx = idx_v[j, pl.ds(K - 16, 16)]
        plsc.addupdate_scatter(tab_v, [idx], one16, mask=tail_mask)
        return 0

    lax.fori_loop(0, CH_PER_TILE, body, 0)
    pltpu.sync_copy(tab_v, deg_sh.at[pl.ds(s * NPAD, NPAD)])
    plsc.subcore_barrier()
    col0 = s * ROWS_PER_TILE
    for k in range(ROWS_PER_TILE // 16):
        res_v[pl.ds(k * 16, 16)] = zero16
    for t in range(NSUB):
        pltpu.sync_copy(deg_sh.at[pl.ds(t * NPAD + col0, ROWS_PER_TILE)], buf_v)

        def acc(m, _):
            for k in range(8):
                i0 = m * 128 + k * 16
                res_v[pl.ds(i0, 16)] = (res_v[pl.ds(i0, 16)]
                                        + buf_v[pl.ds(i0, 16)])
            return 0

        lax.fori_loop(0, ROWS_PER_TILE // 128, acc, 0)
    pltpu.sync_copy(res_v, out_hbm.at[pl.ds(c * NPAD + col0, ROWS_PER_TILE)])


_sc_deg = functools.partial(
    pl.kernel,
    out_type=jax.ShapeDtypeStruct((NCORES * NPAD,), jnp.float32),
    mesh=_mesh,
    scratch_types=[
        pltpu.VMEM((CH_PER_TILE, K), jnp.int32),
        pltpu.VMEM((NPAD,), jnp.float32),
        pltpu.VMEM((ROWS_PER_TILE,), jnp.float32),
        pltpu.VMEM((ROWS_PER_TILE,), jnp.float32),
        pltpu.VMEM_SHARED((NSUB * NPAD,), jnp.float32),
    ],
    compiler_params=pltpu.CompilerParams(needs_layout_passes=False),
)(_sc_deg_body)


NBUF = 2
PHASES = 2
CPP = CH_PER_TILE // PHASES
ROUNDS = CPP // NBUF


def _sc_edge_body(src_hbm, dst_hbm, h_hbm, out_hbm, src_v, dst_v,
                  b0, b1, agg_sh, g0, g1, s0, s1):
    rows = [b0, b1]
    gsem = [g0, g1]
    ssem = [s0, s1]
    c = lax.axis_index("c")
    s = lax.axis_index("s")
    _fill_rows(b0, K, D // 16, 0.0)
    for z in range(5):
        pltpu.sync_copy(b0.at[pl.ds(0, 120)],
                        agg_sh.at[pl.ds(s * ROWS_PER_TILE + z * 120, 120)])
    pltpu.sync_copy(b0.at[pl.ds(0, 40)],
                    agg_sh.at[pl.ds(s * ROWS_PER_TILE + 600, 40)])
    plsc.subcore_barrier()
    base = c * CH_PER_CORE + s * CH_PER_TILE

    def phase(p, _):
        pltpu.sync_copy(src_hbm.at[pl.ds(base + p * CPP, CPP)], src_v)
        pltpu.sync_copy(dst_hbm.at[pl.ds(base + p * CPP, CPP)], dst_v)
        for b in range(NBUF):
            pltpu.async_copy(h_hbm.at[src_v.at[b]], rows[b], gsem[b])

        def rnd(r, _):
            for b in range(NBUF):
                j = r * NBUF + b
                pltpu.make_async_copy(h_hbm.at[src_v.at[0]], rows[b],
                                      gsem[b]).wait()
                pltpu.async_copy(rows[b], agg_sh.at[dst_v.at[j]], ssem[b],
                                 add=True)
            for b in range(NBUF):
                jn = r * NBUF + b + NBUF

                @pl.when(jn < CPP)
                def _(b=b, jn=jn):
                    pltpu.make_async_copy(rows[b], agg_sh.at[dst_v.at[0]],
                                          ssem[b]).wait()
                    pltpu.async_copy(h_hbm.at[src_v.at[jn]], rows[b], gsem[b])
            return 0

        lax.fori_loop(0, ROUNDS, rnd, 0)
        for b in range(NBUF):
            pltpu.make_async_copy(rows[b], agg_sh.at[dst_v.at[0]],
                                  ssem[b]).wait()
        return 0

    lax.fori_loop(0, PHASES, phase, 0)
    plsc.subcore_barrier()
    r0 = s * ROWS_PER_TILE
    pltpu.sync_copy(agg_sh.at[pl.ds(r0, ROWS_PER_TILE)],
                    out_hbm.at[c, pl.ds(r0, ROWS_PER_TILE)])


_sc_edge = functools.partial(
    pl.kernel,
    out_type=jax.ShapeDtypeStruct((NCORES, NPAD, D), jnp.float32),
    mesh=_mesh,
    scratch_types=[
        pltpu.VMEM((CPP, K), jnp.int32),
        pltpu.VMEM((CPP, K), jnp.int32),
    ] + [pltpu.VMEM((K, D), jnp.float32)] * NBUF
      + [pltpu.VMEM_SHARED((NPAD, D), jnp.float32)]
      + [pltpu.SemaphoreType.DMA] * (2 * NBUF),
)(_sc_edge_body)


_BLK = 1000
_GRID = N // _BLK


def _dis(d0, d1):
    return lax.rsqrt(d0[0] + d1[0] + 1.0)


def _tc_pre_body(d0, d1, x, w, o):
    o[...] = _dis(d0, d1) * jnp.dot(x[...], w[...],
                                    preferred_element_type=jnp.float32)


def _tc_mid_body(d0, d1, a0, a1, h, w, o):
    dis = _dis(d0, d1)
    y = jnp.maximum(dis * (a0[0] + a1[0] + h[...]), 0.0)
    o[...] = dis * jnp.dot(y, w[...], preferred_element_type=jnp.float32)


def _tc_post_body(d0, d1, a0, a1, h, o):
    o[...] = _dis(d0, d1) * (a0[0] + a1[0] + h[...])


_d0_spec = pl.BlockSpec((1, _BLK, 1), lambda i: (0, i, 0))
_d1_spec = pl.BlockSpec((1, _BLK, 1), lambda i: (1, i, 0))
_a0_spec = pl.BlockSpec((1, _BLK, D), lambda i: (0, i, 0))
_a1_spec = pl.BlockSpec((1, _BLK, D), lambda i: (1, i, 0))
_row_spec = pl.BlockSpec((_BLK, D), lambda i: (i, 0))
_w_spec = pl.BlockSpec((D, D), lambda i: (0, 0))
_out_sds = jax.ShapeDtypeStruct((NPAD, D), jnp.float32)

_tc_pre = pl.pallas_call(
    _tc_pre_body, grid=(_GRID,),
    in_specs=[_d0_spec, _d1_spec, _row_spec, _w_spec],
    out_specs=_row_spec, out_shape=_out_sds)

_tc_mid = pl.pallas_call(
    _tc_mid_body, grid=(_GRID,),
    in_specs=[_d0_spec, _d1_spec, _a0_spec, _a1_spec, _row_spec, _w_spec],
    out_specs=_row_spec, out_shape=_out_sds)

_tc_post = pl.pallas_call(
    _tc_post_body, grid=(_GRID,),
    in_specs=[_d0_spec, _d1_spec, _a0_spec, _a1_spec, _row_spec],
    out_specs=_row_spec, out_shape=jax.ShapeDtypeStruct((N, D), jnp.float32))


def kernel(x, edge_index, W0, W1, W2):
    src2 = edge_index[0].reshape(CHUNKS, K)
    dst2 = edge_index[1].reshape(CHUNKS, K)

    degp = _sc_deg(dst2).reshape(NCORES, NPAD, 1)
    h = _tc_pre(degp, degp, x, W0)
    a = _sc_edge(src2, dst2, h)
    h = _tc_mid(degp, degp, a, a, h, W1)
    a = _sc_edge(src2, dst2, h)
    h = _tc_mid(degp, degp, a, a, h, W2)
    a = _sc_edge(src2, dst2, h)
    return _tc_post(degp, degp, a, a, h)


# speedup vs baseline: 25.4667x; 1.1195x over previous
import functools

import jax
import jax.numpy as jnp
from jax import lax
from jax.experimental import pallas as pl
from jax.experimental.pallas import tpu as pltpu
from jax.experimental.pallas import tpu_sc as plsc

N = 10000
D = 128
E = 320000

NPAD = 10240
K = 50
CHUNKS = E // K
NCORES = 2
NSUB = 16
CH_PER_CORE = CHUNKS // NCORES
CH_PER_TILE = CH_PER_CORE // NSUB
ROWS_PER_TILE = NPAD // NSUB

_mesh = plsc.VectorSubcoreMesh(core_axis_name="c", subcore_axis_name="s")


def _fill_rows(ref, nrows, ncolchunks, value):
    v = jnp.full((16,), value, jnp.float32)

    def outer(r, _):
        for k in range(ncolchunks):
            ref[r, pl.ds(k * 16, 16)] = v
        return 0

    lax.fori_loop(0, nrows, outer, 0)


def _sc_deg_body(dst_hbm, out_hbm, idx_v, tab_v, buf_v, res_v, deg_sh):
    c = lax.axis_index("c")
    s = lax.axis_index("s")
    zero16 = jnp.zeros((16,), jnp.float32)
    one16 = jnp.ones((16,), jnp.float32)

    def z(i, _):
        for k in range(16):
            tab_v[pl.ds(i * 256 + k * 16, 16)] = zero16
        return 0

    lax.fori_loop(0, NPAD // 256, z, 0)
    base = c * CH_PER_CORE + s * CH_PER_TILE
    pltpu.sync_copy(dst_hbm.at[pl.ds(base, CH_PER_TILE)], idx_v)

    tail_mask = lax.iota(jnp.int32, 16) >= (16 - (K - (K // 16) * 16))

    def body(j, _):
        for k in range(K // 16):
            idx = idx_v[j, pl.ds(k * 16, 16)]
            plsc.addupdate_scatter(tab_v, [idx], one16)
        idx = idx_v[j, pl.ds(K - 16, 16)]
        plsc.addupdate_scatter(tab_v, [idx], one16, mask=tail_mask)
        return 0

    lax.fori_loop(0, CH_PER_TILE, body, 0)
    pltpu.sync_copy(tab_v, deg_sh.at[pl.ds(s * NPAD, NPAD)])
    plsc.subcore_barrier()
    col0 = s * ROWS_PER_TILE
    for k in range(ROWS_PER_TILE // 16):
        res_v[pl.ds(k * 16, 16)] = zero16
    for t in range(NSUB):
        pltpu.sync_copy(deg_sh.at[pl.ds(t * NPAD + col0, ROWS_PER_TILE)], buf_v)

        def acc(m, _):
            for k in range(8):
                i0 = m * 128 + k * 16
                res_v[pl.ds(i0, 16)] = (res_v[pl.ds(i0, 16)]
                                        + buf_v[pl.ds(i0, 16)])
            return 0

        lax.fori_loop(0, ROWS_PER_TILE // 128, acc, 0)
    pltpu.sync_copy(res_v, out_hbm.at[pl.ds(c * NPAD + col0, ROWS_PER_TILE)])


_sc_deg = functools.partial(
    pl.kernel,
    out_type=jax.ShapeDtypeStruct((NCORES * NPAD,), jnp.float32),
    mesh=_mesh,
    scratch_types=[
        pltpu.VMEM((CH_PER_TILE, K), jnp.int32),
        pltpu.VMEM((NPAD,), jnp.float32),
        pltpu.VMEM((ROWS_PER_TILE,), jnp.float32),
        pltpu.VMEM((ROWS_PER_TILE,), jnp.float32),
        pltpu.VMEM_SHARED((NSUB * NPAD,), jnp.float32),
    ],
    compiler_params=pltpu.CompilerParams(needs_layout_passes=False),
)(_sc_deg_body)


NBUF = 4
PHASES = 5
CPP = CH_PER_TILE // PHASES
ROUNDS = CPP // NBUF


def _sc_edge_body(src_hbm, dst_hbm, h_hbm, out_hbm, src_v, dst_v,
                  b0, b1, b2, b3, agg_sh, g0, g1, g2, g3, s0, s1, s2, s3):
    rows = [b0, b1, b2, b3]
    gsem = [g0, g1, g2, g3]
    ssem = [s0, s1, s2, s3]
    c = lax.axis_index("c")
    s = lax.axis_index("s")
    _fill_rows(b0, K, D // 16, 0.0)
    for z in range(16):
        pltpu.sync_copy(b0.at[pl.ds(0, 40)],
                        agg_sh.at[pl.ds(s * ROWS_PER_TILE + z * 40, 40)])
    plsc.subcore_barrier()
    base = c * CH_PER_CORE + s * CH_PER_TILE

    def phase(p, _):
        pltpu.sync_copy(src_hbm.at[pl.ds(base + p * CPP, CPP)], src_v)
        pltpu.sync_copy(dst_hbm.at[pl.ds(base + p * CPP, CPP)], dst_v)
        for b in range(NBUF):
            pltpu.async_copy(h_hbm.at[src_v.at[b]], rows[b], gsem[b])

        def rnd(r, _):
            for b in range(NBUF):
                j = r * NBUF + b
                pltpu.make_async_copy(h_hbm.at[src_v.at[0]], rows[b],
                                      gsem[b]).wait()
                pltpu.async_copy(rows[b], agg_sh.at[dst_v.at[j]], ssem[b],
                                 add=True)
            for b in range(NBUF):
                jn = r * NBUF + b + NBUF

                @pl.when(jn < CPP)
                def _(b=b, jn=jn):
                    pltpu.make_async_copy(rows[b], agg_sh.at[dst_v.at[0]],
                                          ssem[b]).wait()
                    pltpu.async_copy(h_hbm.at[src_v.at[jn]], rows[b], gsem[b])
            return 0

        lax.fori_loop(0, ROUNDS, rnd, 0)
        for b in range(NBUF):
            pltpu.make_async_copy(rows[b], agg_sh.at[dst_v.at[0]],
                                  ssem[b]).wait()
        return 0

    lax.fori_loop(0, PHASES, phase, 0)
    plsc.subcore_barrier()
    r0 = s * ROWS_PER_TILE
    pltpu.sync_copy(agg_sh.at[pl.ds(r0, ROWS_PER_TILE)],
                    out_hbm.at[c, pl.ds(r0, ROWS_PER_TILE)])


_sc_edge = functools.partial(
    pl.kernel,
    out_type=jax.ShapeDtypeStruct((NCORES, NPAD, D), jnp.float32),
    mesh=_mesh,
    scratch_types=[
        pltpu.VMEM((CPP, K), jnp.int32),
        pltpu.VMEM((CPP, K), jnp.int32),
    ] + [pltpu.VMEM((K, D), jnp.float32)] * NBUF
      + [pltpu.VMEM_SHARED((NPAD, D), jnp.float32)]
      + [pltpu.SemaphoreType.DMA] * (2 * NBUF),
)(_sc_edge_body)


_BLK = 1000
_GRID = N // _BLK


def _dis(d0, d1):
    return lax.rsqrt(d0[0] + d1[0] + 1.0)


def _tc_pre_body(d0, d1, x, w, o):
    o[...] = _dis(d0, d1) * jnp.dot(x[...], w[...],
                                    preferred_element_type=jnp.float32)


def _tc_mid_body(d0, d1, a0, a1, h, w, o):
    dis = _dis(d0, d1)
    y = jnp.maximum(dis * (a0[0] + a1[0] + h[...]), 0.0)
    o[...] = dis * jnp.dot(y, w[...], preferred_element_type=jnp.float32)


def _tc_post_body(d0, d1, a0, a1, h, o):
    o[...] = _dis(d0, d1) * (a0[0] + a1[0] + h[...])


_d0_spec = pl.BlockSpec((1, _BLK, 1), lambda i: (0, i, 0))
_d1_spec = pl.BlockSpec((1, _BLK, 1), lambda i: (1, i, 0))
_a0_spec = pl.BlockSpec((1, _BLK, D), lambda i: (0, i, 0))
_a1_spec = pl.BlockSpec((1, _BLK, D), lambda i: (1, i, 0))
_row_spec = pl.BlockSpec((_BLK, D), lambda i: (i, 0))
_w_spec = pl.BlockSpec((D, D), lambda i: (0, 0))
_out_sds = jax.ShapeDtypeStruct((NPAD, D), jnp.float32)

_tc_pre = pl.pallas_call(
    _tc_pre_body, grid=(_GRID,),
    in_specs=[_d0_spec, _d1_spec, _row_spec, _w_spec],
    out_specs=_row_spec, out_shape=_out_sds)

_tc_mid = pl.pallas_call(
    _tc_mid_body, grid=(_GRID,),
    in_specs=[_d0_spec, _d1_spec, _a0_spec, _a1_spec, _row_spec, _w_spec],
    out_specs=_row_spec, out_shape=_out_sds)

_tc_post = pl.pallas_call(
    _tc_post_body, grid=(_GRID,),
    in_specs=[_d0_spec, _d1_spec, _a0_spec, _a1_spec, _row_spec],
    out_specs=_row_spec, out_shape=jax.ShapeDtypeStruct((N, D), jnp.float32))


def kernel(x, edge_index, W0, W1, W2):
    src2 = edge_index[0].reshape(CHUNKS, K)
    dst2 = edge_index[1].reshape(CHUNKS, K)

    degp = _sc_deg(dst2).reshape(NCORES, NPAD, 1)
    h = _tc_pre(degp, degp, x, W0)
    a = _sc_edge(src2, dst2, h)
    h = _tc_mid(degp, degp, a, a, h, W1)
    a = _sc_edge(src2, dst2, h)
    h = _tc_mid(degp, degp, a, a, h, W2)
    a = _sc_edge(src2, dst2, h)
    return _tc_post(degp, degp, a, a, h)


# speedup vs baseline: 26.2488x; 1.0307x over previous
import functools

import jax
import jax.numpy as jnp
from jax import lax
from jax.experimental import pallas as pl
from jax.experimental.pallas import tpu as pltpu
from jax.experimental.pallas import tpu_sc as plsc

N = 10000
D = 128
E = 320000

NPAD = 10240
K = 50
CHUNKS = E // K
NCORES = 2
NSUB = 16
CH_PER_CORE = CHUNKS // NCORES
CH_PER_TILE = CH_PER_CORE // NSUB
ROWS_PER_TILE = NPAD // NSUB

_mesh = plsc.VectorSubcoreMesh(core_axis_name="c", subcore_axis_name="s")


def _fill_rows(ref, nrows, ncolchunks, value):
    v = jnp.full((16,), value, jnp.float32)

    def outer(r, _):
        for k in range(ncolchunks):
            ref[r, pl.ds(k * 16, 16)] = v
        return 0

    lax.fori_loop(0, nrows, outer, 0)


def _sc_deg_body(e2_hbm, out_hbm, idx_v, tab_v, buf_v, res_v, deg_sh):
    dst_hbm = e2_hbm.at[1]
    c = lax.axis_index("c")
    s = lax.axis_index("s")
    zero16 = jnp.zeros((16,), jnp.float32)
    one16 = jnp.ones((16,), jnp.float32)

    def z(i, _):
        for k in range(16):
            tab_v[pl.ds(i * 256 + k * 16, 16)] = zero16
        return 0

    lax.fori_loop(0, NPAD // 256, z, 0)
    base = c * CH_PER_CORE + s * CH_PER_TILE
    pltpu.sync_copy(dst_hbm.at[pl.ds(base, CH_PER_TILE)], idx_v)

    tail_mask = lax.iota(jnp.int32, 16) >= (16 - (K - (K // 16) * 16))

    def body(j, _):
        for k in range(K // 16):
            idx = idx_v[j, pl.ds(k * 16, 16)]
            plsc.addupdate_scatter(tab_v, [idx], one16)
        idx = idx_v[j, pl.ds(K - 16, 16)]
        plsc.addupdate_scatter(tab_v, [idx], one16, mask=tail_mask)
        return 0

    lax.fori_loop(0, CH_PER_TILE, body, 0)
    pltpu.sync_copy(tab_v, deg_sh.at[pl.ds(s * NPAD, NPAD)])
    plsc.subcore_barrier()
    col0 = s * ROWS_PER_TILE
    for k in range(ROWS_PER_TILE // 16):
        res_v[pl.ds(k * 16, 16)] = zero16
    for t in range(NSUB):
        pltpu.sync_copy(deg_sh.at[pl.ds(t * NPAD + col0, ROWS_PER_TILE)], buf_v)

        def acc(m, _):
            for k in range(8):
                i0 = m * 128 + k * 16
                res_v[pl.ds(i0, 16)] = (res_v[pl.ds(i0, 16)]
                                        + buf_v[pl.ds(i0, 16)])
            return 0

        lax.fori_loop(0, ROWS_PER_TILE // 128, acc, 0)
    pltpu.sync_copy(res_v, out_hbm.at[pl.ds(c * NPAD + col0, ROWS_PER_TILE)])


_sc_deg = functools.partial(
    pl.kernel,
    out_type=jax.ShapeDtypeStruct((NCORES * NPAD,), jnp.float32),
    mesh=_mesh,
    scratch_types=[
        pltpu.VMEM((CH_PER_TILE, K), jnp.int32),
        pltpu.VMEM((NPAD,), jnp.float32),
        pltpu.VMEM((ROWS_PER_TILE,), jnp.float32),
        pltpu.VMEM((ROWS_PER_TILE,), jnp.float32),
        pltpu.VMEM_SHARED((NSUB * NPAD,), jnp.float32),
    ],
    compiler_params=pltpu.CompilerParams(needs_layout_passes=False),
)(_sc_deg_body)


NBUF = 4
PHASES = 5
CPP = CH_PER_TILE // PHASES
ROUNDS = CPP // NBUF


def _sc_edge_body(e2_hbm, h_hbm, out_hbm, src_v, dst_v,
                  b0, b1, b2, b3, agg_sh, g0, g1, g2, g3, s0, s1, s2, s3):
    src_hbm = e2_hbm.at[0]
    dst_hbm = e2_hbm.at[1]
    rows = [b0, b1, b2, b3]
    gsem = [g0, g1, g2, g3]
    ssem = [s0, s1, s2, s3]
    c = lax.axis_index("c")
    s = lax.axis_index("s")
    _fill_rows(b0, K, D // 16, 0.0)
    for z in range(16):
        pltpu.sync_copy(b0.at[pl.ds(0, 40)],
                        agg_sh.at[pl.ds(s * ROWS_PER_TILE + z * 40, 40)])
    plsc.subcore_barrier()
    base = c * CH_PER_CORE + s * CH_PER_TILE

    def phase(p, _):
        pltpu.sync_copy(src_hbm.at[pl.ds(base + p * CPP, CPP)], src_v)
        pltpu.sync_copy(dst_hbm.at[pl.ds(base + p * CPP, CPP)], dst_v)
        for b in range(NBUF):
            pltpu.async_copy(h_hbm.at[src_v.at[b]], rows[b], gsem[b])

        def rnd(r, _):
            for b in range(NBUF):
                j = r * NBUF + b
                pltpu.make_async_copy(h_hbm.at[src_v.at[0]], rows[b],
                                      gsem[b]).wait()
                pltpu.async_copy(rows[b], agg_sh.at[dst_v.at[j]], ssem[b],
                                 add=True)
            for b in range(NBUF):
                jn = r * NBUF + b + NBUF

                @pl.when(jn < CPP)
                def _(b=b, jn=jn):
                    pltpu.make_async_copy(rows[b], agg_sh.at[dst_v.at[0]],
                                          ssem[b]).wait()
                    pltpu.async_copy(h_hbm.at[src_v.at[jn]], rows[b], gsem[b])
            return 0

        lax.fori_loop(0, ROUNDS, rnd, 0)
        for b in range(NBUF):
            pltpu.make_async_copy(rows[b], agg_sh.at[dst_v.at[0]],
                                  ssem[b]).wait()
        return 0

    lax.fori_loop(0, PHASES, phase, 0)
    plsc.subcore_barrier()
    r0 = s * ROWS_PER_TILE
    pltpu.sync_copy(agg_sh.at[pl.ds(r0, ROWS_PER_TILE)],
                    out_hbm.at[c, pl.ds(r0, ROWS_PER_TILE)])


_sc_edge = functools.partial(
    pl.kernel,
    out_type=jax.ShapeDtypeStruct((NCORES, NPAD, D), jnp.float32),
    mesh=_mesh,
    scratch_types=[
        pltpu.VMEM((CPP, K), jnp.int32),
        pltpu.VMEM((CPP, K), jnp.int32),
    ] + [pltpu.VMEM((K, D), jnp.float32)] * NBUF
      + [pltpu.VMEM_SHARED((NPAD, D), jnp.float32)]
      + [pltpu.SemaphoreType.DMA] * (2 * NBUF),
)(_sc_edge_body)


_BLK = 2048
_GRID = NPAD // _BLK
_DROWS = _BLK // 128


def _dis(d0, d1):
    return lax.rsqrt(d0[0] + d1[0] + 1.0)


def _tc_pre_body(d0, d1, x, w, o):
    o[...] = _dis(d0, d1) * jnp.dot(x[...], w[...],
                                    preferred_element_type=jnp.float32)


def _tc_mid_body(d0, d1, a0, a1, h, w, o):
    dis = _dis(d0, d1)
    y = jnp.maximum(dis * (a0[0] + a1[0] + h[...]), 0.0)
    o[...] = dis * jnp.dot(y, w[...], preferred_element_type=jnp.float32)


def _tc_post_body(d0, d1, a0, a1, h, o):
    o[...] = _dis(d0, d1) * (a0[0] + a1[0] + h[...])


_d0_spec = pl.BlockSpec((1, _BLK, 1), lambda i: (0, i, 0))
_d1_spec = pl.BlockSpec((1, _BLK, 1), lambda i: (1, i, 0))
_a0_spec = pl.BlockSpec((1, _BLK, D), lambda i: (0, i, 0))
_a1_spec = pl.BlockSpec((1, _BLK, D), lambda i: (1, i, 0))
_row_spec = pl.BlockSpec((_BLK, D), lambda i: (i, 0))
_w_spec = pl.BlockSpec((D, D), lambda i: (0, 0))
_out_sds = jax.ShapeDtypeStruct((NPAD, D), jnp.float32)

_tc_pre = pl.pallas_call(
    _tc_pre_body, grid=(_GRID,),
    in_specs=[_d0_spec, _d1_spec, _row_spec, _w_spec],
    out_specs=_row_spec, out_shape=_out_sds)

_tc_mid = pl.pallas_call(
    _tc_mid_body, grid=(_GRID,),
    in_specs=[_d0_spec, _d1_spec, _a0_spec, _a1_spec, _row_spec, _w_spec],
    out_specs=_row_spec, out_shape=_out_sds)

_tc_post = pl.pallas_call(
    _tc_post_body, grid=(_GRID,),
    in_specs=[_d0_spec, _d1_spec, _a0_spec, _a1_spec, _row_spec],
    out_specs=_row_spec, out_shape=jax.ShapeDtypeStruct((N, D), jnp.float32))


def kernel(x, edge_index, W0, W1, W2):
    e2 = edge_index.reshape(NCORES, CHUNKS, K)

    degp = _sc_deg(e2).reshape(NCORES, NPAD, 1)
    h = _tc_pre(degp, degp, x, W0)
    a = _sc_edge(e2, h)
    h = _tc_mid(degp, degp, a, a, h, W1)
    a = _sc_edge(e2, h)
    h = _tc_mid(degp, degp, a, a, h, W2)
    a = _sc_edge(e2, h)
    return _tc_post(degp, degp, a, a, h)


# speedup vs baseline: 26.4429x; 1.0074x over previous
import functools

import jax
import jax.numpy as jnp
from jax import lax
from jax.experimental import pallas as pl
from jax.experimental.pallas import tpu as pltpu
from jax.experimental.pallas import tpu_sc as plsc

N = 10000
D = 128
E = 320000

NPAD = 10240
K = 50
CHUNKS = E // K
NCORES = 2
NSUB = 16
CH_PER_CORE = CHUNKS // NCORES
CH_PER_TILE = CH_PER_CORE // NSUB
ROWS_PER_TILE = NPAD // NSUB

_mesh = plsc.VectorSubcoreMesh(core_axis_name="c", subcore_axis_name="s")


def _fill_rows(ref, nrows, ncolchunks, value):
    v = jnp.full((16,), value, jnp.float32)

    def outer(r, _):
        for k in range(ncolchunks):
            ref[r, pl.ds(k * 16, 16)] = v
        return 0

    lax.fori_loop(0, nrows, outer, 0)


def _sc_deg_body(e2_hbm, out_hbm, idx_v, tab_v, buf_v, res_v, deg_sh):
    dst_hbm = e2_hbm.at[1]
    c = lax.axis_index("c")
    s = lax.axis_index("s")
    zero16 = jnp.zeros((16,), jnp.float32)
    one16 = jnp.ones((16,), jnp.float32)

    def z(i, _):
        for k in range(16):
            tab_v[pl.ds(i * 256 + k * 16, 16)] = zero16
        return 0

    lax.fori_loop(0, NPAD // 256, z, 0)
    base = c * CH_PER_CORE + s * CH_PER_TILE
    pltpu.sync_copy(dst_hbm.at[pl.ds(base, CH_PER_TILE)], idx_v)

    tail_mask = lax.iota(jnp.int32, 16) >= (16 - (K - (K // 16) * 16))

    def body(j, _):
        for k in range(K // 16):
            idx = idx_v[j, pl.ds(k * 16, 16)]
            plsc.addupdate_scatter(tab_v, [idx], one16)
        idx = idx_v[j, pl.ds(K - 16, 16)]
        plsc.addupdate_scatter(tab_v, [idx], one16, mask=tail_mask)
        return 0

    lax.fori_loop(0, CH_PER_TILE, body, 0)
    pltpu.sync_copy(tab_v, deg_sh.at[pl.ds(s * NPAD, NPAD)])
    plsc.subcore_barrier()
    col0 = s * ROWS_PER_TILE
    for k in range(ROWS_PER_TILE // 16):
        res_v[pl.ds(k * 16, 16)] = zero16
    for t in range(NSUB):
        pltpu.sync_copy(deg_sh.at[pl.ds(t * NPAD + col0, ROWS_PER_TILE)], buf_v)

        def acc(m, _):
            for k in range(8):
                i0 = m * 128 + k * 16
                res_v[pl.ds(i0, 16)] = (res_v[pl.ds(i0, 16)]
                                        + buf_v[pl.ds(i0, 16)])
            return 0

        lax.fori_loop(0, ROWS_PER_TILE // 128, acc, 0)
    pltpu.sync_copy(res_v, out_hbm.at[pl.ds(c * NPAD + col0, ROWS_PER_TILE)])


_sc_deg = functools.partial(
    pl.kernel,
    out_type=jax.ShapeDtypeStruct((NCORES * NPAD,), jnp.float32),
    mesh=_mesh,
    scratch_types=[
        pltpu.VMEM((CH_PER_TILE, K), jnp.int32),
        pltpu.VMEM((NPAD,), jnp.float32),
        pltpu.VMEM((ROWS_PER_TILE,), jnp.float32),
        pltpu.VMEM((ROWS_PER_TILE,), jnp.float32),
        pltpu.VMEM_SHARED((NSUB * NPAD,), jnp.float32),
    ],
    compiler_params=pltpu.CompilerParams(needs_layout_passes=False),
)(_sc_deg_body)


NBUF = 4
PHASES = 5
CPP = CH_PER_TILE // PHASES
ROUNDS = CPP // NBUF


def _sc_edge_body(e2_hbm, h_hbm, out_hbm, src_v, dst_v,
                  b0, b1, b2, b3, agg_sh, g0, g1, g2, g3, s0, s1, s2, s3):
    src_hbm = e2_hbm.at[0]
    dst_hbm = e2_hbm.at[1]
    rows = [b0, b1, b2, b3]
    gsem = [g0, g1, g2, g3]
    ssem = [s0, s1, s2, s3]
    c = lax.axis_index("c")
    s = lax.axis_index("s")
    _fill_rows(b0, K, D // 16, 0.0)
    for z in range(16):
        pltpu.sync_copy(b0.at[pl.ds(0, 40)],
                        agg_sh.at[pl.ds(s * ROWS_PER_TILE + z * 40, 40)])
    plsc.subcore_barrier()
    base = c * CH_PER_CORE + s * CH_PER_TILE

    def phase(p, _):
        pltpu.sync_copy(src_hbm.at[pl.ds(base + p * CPP, CPP)], src_v)
        pltpu.sync_copy(dst_hbm.at[pl.ds(base + p * CPP, CPP)], dst_v)
        for b in range(NBUF):
            pltpu.async_copy(h_hbm.at[src_v.at[b]], rows[b], gsem[b])

        def rnd(r, _):
            for b in range(NBUF):
                j = r * NBUF + b
                pltpu.make_async_copy(h_hbm.at[src_v.at[0]], rows[b],
                                      gsem[b]).wait()
                pltpu.async_copy(rows[b], agg_sh.at[dst_v.at[j]], ssem[b],
                                 add=True)
            for b in range(NBUF):
                jn = r * NBUF + b + NBUF

                @pl.when(jn < CPP)
                def _(b=b, jn=jn):
                    pltpu.make_async_copy(rows[b], agg_sh.at[dst_v.at[0]],
                                          ssem[b]).wait()
                    pltpu.async_copy(h_hbm.at[src_v.at[jn]], rows[b], gsem[b])
            return 0

        lax.fori_loop(0, ROUNDS, rnd, 0)
        for b in range(NBUF):
            pltpu.make_async_copy(rows[b], agg_sh.at[dst_v.at[0]],
                                  ssem[b]).wait()
        return 0

    lax.fori_loop(0, PHASES, phase, 0)
    plsc.subcore_barrier()
    r0 = s * ROWS_PER_TILE
    pltpu.sync_copy(agg_sh.at[pl.ds(r0, ROWS_PER_TILE)],
                    out_hbm.at[c, pl.ds(r0, ROWS_PER_TILE)])


_sc_edge = functools.partial(
    pl.kernel,
    out_type=jax.ShapeDtypeStruct((NCORES, NPAD, D), jnp.float32),
    mesh=_mesh,
    scratch_types=[
        pltpu.VMEM((CPP, K), jnp.int32),
        pltpu.VMEM((CPP, K), jnp.int32),
    ] + [pltpu.VMEM((K, D), jnp.float32)] * NBUF
      + [pltpu.VMEM_SHARED((NPAD, D), jnp.float32)]
      + [pltpu.SemaphoreType.DMA] * (2 * NBUF),
)(_sc_edge_body)


_BLK = 2048
_GRID = NPAD // _BLK
_DROWS = _BLK // 128


def _dis(d0, d1):
    return lax.rsqrt(d0[0] + d1[0] + 1.0)


def _tc_pre_body(d0, d1, x, w, o, odis):
    dis = _dis(d0, d1)
    odis[...] = dis
    o[...] = dis * jnp.dot(x[...], w[...], preferred_element_type=jnp.float32)


def _tc_mid_body(dis_r, a0, a1, h, w, o):
    dis = dis_r[...]
    y = jnp.maximum(dis * (a0[0] + a1[0] + h[...]), 0.0)
    o[...] = dis * jnp.dot(y, w[...], preferred_element_type=jnp.float32)


def _tc_post_body(dis_r, a0, a1, h, o):
    o[...] = dis_r[...] * (a0[0] + a1[0] + h[...])


_d0_spec = pl.BlockSpec((1, _BLK, 1), lambda i: (0, i, 0))
_d1_spec = pl.BlockSpec((1, _BLK, 1), lambda i: (1, i, 0))
_a0_spec = pl.BlockSpec((1, _BLK, D), lambda i: (0, i, 0))
_a1_spec = pl.BlockSpec((1, _BLK, D), lambda i: (1, i, 0))
_row_spec = pl.BlockSpec((_BLK, D), lambda i: (i, 0))
_w_spec = pl.BlockSpec((D, D), lambda i: (0, 0))
_out_sds = jax.ShapeDtypeStruct((NPAD, D), jnp.float32)

_dis_spec = pl.BlockSpec((_BLK, 1), lambda i: (i, 0))
_dis_sds = jax.ShapeDtypeStruct((NPAD, 1), jnp.float32)

_tc_pre = pl.pallas_call(
    _tc_pre_body, grid=(_GRID,),
    in_specs=[_d0_spec, _d1_spec, _row_spec, _w_spec],
    out_specs=(_row_spec, _dis_spec), out_shape=(_out_sds, _dis_sds))

_tc_mid = pl.pallas_call(
    _tc_mid_body, grid=(_GRID,),
    in_specs=[_dis_spec, _a0_spec, _a1_spec, _row_spec, _w_spec],
    out_specs=_row_spec, out_shape=_out_sds)

_tc_post = pl.pallas_call(
    _tc_post_body, grid=(_GRID,),
    in_specs=[_dis_spec, _a0_spec, _a1_spec, _row_spec],
    out_specs=_row_spec, out_shape=jax.ShapeDtypeStruct((N, D), jnp.float32))


def kernel(x, edge_index, W0, W1, W2):
    e2 = edge_index.reshape(NCORES, CHUNKS, K)

    degp = _sc_deg(e2).reshape(NCORES, NPAD, 1)
    h, dis = _tc_pre(degp, degp, x, W0)
    a = _sc_edge(e2, h)
    h = _tc_mid(dis, a, a, h, W1)
    a = _sc_edge(e2, h)
    h = _tc_mid(dis, a, a, h, W2)
    a = _sc_edge(e2, h)
    return _tc_post(dis, a, a, h)
